# Initial kernel scaffold; baseline (speedup 1.0000x reference)
import jax
import jax.numpy as jnp
from jax.experimental import pallas as pl


def kernel(x_base, x_pa, mobility, ply, W1b, b1b, W1pa, b1pa, W2, b2, Wout, bout):
    raise NotImplementedError("write your pallas kernel here")


# speedup vs baseline: 19.4269x; 19.4269x over previous
import jax
import jax.numpy as jnp
from jax import lax
from jax.experimental import pallas as pl

_COUNT = 8
_B = 16384
_C = 255.0 / 256.0
_BM = 1024


def _ls_kernel(xb_ref, xpa_ref, mob_ref, ply_ref,
               w1x_ref, w1m_ref, b1_ref,
               w2_ref, b2_ref,
               wol2_ref, woxb_ref, woxpa_ref, bo_ref,
               out_ref):
    xb = xb_ref[...]
    xpa = xpa_ref[...]
    mob = mob_ref[...]
    ply = ply_ref[...]

    xm = jnp.minimum(mob * (7.0 / 255.0), 1.0)

    h1b = jnp.dot(xb, w1x_ref[:, :64], preferred_element_type=jnp.float32)
    h1pa = jnp.dot(xpa, w1x_ref[:, 64:], preferred_element_type=jnp.float32)
    h = jnp.concatenate([h1b, h1pa], axis=1)
    h = h + xm * w1m_ref[...] + b1_ref[...]

    sq = jnp.minimum(h * h * _C, 1.0)
    lin = jnp.clip(h, 0.0, 1.0)
    z = jnp.concatenate([sq, lin], axis=1)

    l2 = jnp.dot(z, w2_ref[...], preferred_element_type=jnp.float32)
    l2 = l2 + b2_ref[...]
    g = jnp.clip(l2, 0.0, 1.0)
    g = g * g * _C

    o = jnp.dot(g, wol2_ref[...], preferred_element_type=jnp.float32)
    o = o + jnp.dot(xb, woxb_ref[...], preferred_element_type=jnp.float32)
    o = o + jnp.dot(xpa, woxpa_ref[...], preferred_element_type=jnp.float32)
    o = o + bo_ref[...]

    bucket = ply // 7
    lanes = lax.broadcasted_iota(jnp.int32, o.shape, 1)
    sel = jnp.where(lanes == bucket, o, 0.0)
    out_ref[...] = jnp.sum(sel, axis=1, keepdims=True)


def kernel(x_base, x_pa, mobility, ply, W1b, b1b, W1pa, b1pa, W2, b2, Wout, bout):
    f32 = jnp.float32
    eye = jnp.eye(_COUNT, dtype=f32)

    w1bT = jnp.transpose(W1b, (2, 0, 1)).reshape(129, 64)
    w1paT = jnp.transpose(W1pa, (2, 0, 1)).reshape(129, 64)
    w1x = jnp.concatenate([w1bT[:128], w1paT[:128]], axis=1)
    w1m = jnp.concatenate([w1bT[128:], w1paT[128:]], axis=1)
    b1 = jnp.concatenate([b1b.reshape(1, 64), b1pa.reshape(1, 64)], axis=1)

    w2r = jnp.transpose(W2, (0, 2, 1))
    blocks = []
    for g in range(4):
        m = w2r[:, g * 8:(g + 1) * 8, :]
        bd = (eye[:, None, :, None] * m[:, :, None, :]).reshape(64, 512)
        blocks.append(bd)
    w2big = jnp.concatenate(blocks, axis=0)
    b2row = b2.reshape(1, 512)

    wo = Wout[:, 0, :]
    wol2 = (eye[:, None, :] * wo[:, None, :64].transpose(0, 2, 1)).reshape(512, 8)
    woxb = wo[:, 64:192].T
    woxpa = wo[:, 192:320].T
    borow = bout.reshape(1, 8)

    ply2 = ply.reshape(_B, 1).astype(jnp.int32)

    nb = _B // _BM
    grid = (nb,)
    bspec = lambda bs, im: pl.BlockSpec(bs, im)
    row = lambda i: (i, 0)
    full = lambda i: (0, 0)

    out = pl.pallas_call(
        _ls_kernel,
        grid=grid,
        in_specs=[
            bspec((_BM, 128), row),
            bspec((_BM, 128), row),
            bspec((_BM, 1), row),
            bspec((_BM, 1), row),
            bspec((128, 128), full),
            bspec((1, 128), full),
            bspec((1, 128), full),
            bspec((256, 512), full),
            bspec((1, 512), full),
            bspec((512, 8), full),
            bspec((128, 8), full),
            bspec((128, 8), full),
            bspec((1, 8), full),
        ],
        out_specs=bspec((_BM, 1), row),
        out_shape=jax.ShapeDtypeStruct((_B, 1), f32),
    )(x_base, x_pa, mobility, ply2,
      w1x, w1m, b1, w2big, b2row, wol2, woxb, woxpa, borow)
    return out


# speedup vs baseline: 20.8126x; 1.0713x over previous
import jax
import jax.numpy as jnp
from jax import lax
from jax.experimental import pallas as pl
from jax.experimental.pallas import tpu as pltpu

_COUNT = 8
_B = 16384
_C = 255.0 / 256.0
_BM = 2048


def _ls_kernel(xb_ref, xpa_ref, mob_ref, ply_ref,
               w1x_ref, w1m_ref, b1_ref,
               w2_ref, b2_ref,
               wol2_ref, woxb_ref, woxpa_ref, bo_ref,
               out_ref):
    xb = xb_ref[...]
    xpa = xpa_ref[...]
    mob = mob_ref[...]
    ply = ply_ref[...]

    xm = jnp.minimum(mob * (7.0 / 255.0), 1.0)

    h1b = jnp.dot(xb, w1x_ref[:, :64], preferred_element_type=jnp.float32)
    h1pa = jnp.dot(xpa, w1x_ref[:, 64:], preferred_element_type=jnp.float32)
    h = jnp.concatenate([h1b, h1pa], axis=1)
    h = h + xm * w1m_ref[...] + b1_ref[...]

    sq = jnp.minimum(h * h * _C, 1.0)
    lin = jnp.clip(h, 0.0, 1.0)
    z = jnp.concatenate([sq, lin], axis=1)

    l2 = jnp.dot(z, w2_ref[...], preferred_element_type=jnp.float32)
    l2 = l2 + b2_ref[...]
    g = jnp.clip(l2, 0.0, 1.0)
    g = g * g * _C

    o = jnp.dot(g, wol2_ref[...], preferred_element_type=jnp.float32)
    o = o + jnp.dot(xb, woxb_ref[...], preferred_element_type=jnp.float32)
    o = o + jnp.dot(xpa, woxpa_ref[...], preferred_element_type=jnp.float32)
    o = o + bo_ref[...]

    bucket = ply // 7
    lanes = lax.broadcasted_iota(jnp.int32, o.shape, 1)
    sel = jnp.where(lanes == bucket, o, 0.0)
    out_ref[...] = jnp.sum(sel, axis=1, keepdims=True)


def kernel(x_base, x_pa, mobility, ply, W1b, b1b, W1pa, b1pa, W2, b2, Wout, bout):
    f32 = jnp.float32
    eye = jnp.eye(_COUNT, dtype=f32)

    w1bT = jnp.transpose(W1b, (2, 0, 1)).reshape(129, 64)
    w1paT = jnp.transpose(W1pa, (2, 0, 1)).reshape(129, 64)
    w1x = jnp.concatenate([w1bT[:128], w1paT[:128]], axis=1)
    w1m = jnp.concatenate([w1bT[128:], w1paT[128:]], axis=1)
    b1 = jnp.concatenate([b1b.reshape(1, 64), b1pa.reshape(1, 64)], axis=1)

    w2r = jnp.transpose(W2, (0, 2, 1))
    blocks = []
    for g in range(4):
        m = w2r[:, g * 8:(g + 1) * 8, :]
        bd = (eye[:, None, :, None] * m[:, :, None, :]).reshape(64, 512)
        blocks.append(bd)
    w2big = jnp.concatenate(blocks, axis=0)
    b2row = b2.reshape(1, 512)

    wo = Wout[:, 0, :]
    wol2 = (eye[:, None, :] * wo[:, None, :64].transpose(0, 2, 1)).reshape(512, 8)
    woxb = wo[:, 64:192].T
    woxpa = wo[:, 192:320].T
    borow = bout.reshape(1, 8)

    ply2 = ply.reshape(_B, 1).astype(jnp.int32)

    nb = _B // _BM
    grid = (nb,)
    bspec = lambda bs, im: pl.BlockSpec(bs, im)
    row = lambda i: (i, 0)
    full = lambda i: (0, 0)

    out = pl.pallas_call(
        _ls_kernel,
        grid=grid,
        in_specs=[
            bspec((_BM, 128), row),
            bspec((_BM, 128), row),
            bspec((_BM, 1), row),
            bspec((_BM, 1), row),
            bspec((128, 128), full),
            bspec((1, 128), full),
            bspec((1, 128), full),
            bspec((256, 512), full),
            bspec((1, 512), full),
            bspec((512, 8), full),
            bspec((128, 8), full),
            bspec((128, 8), full),
            bspec((1, 8), full),
        ],
        out_specs=bspec((_BM, 1), row),
        out_shape=jax.ShapeDtypeStruct((_B, 1), f32),
        compiler_params=pltpu.CompilerParams(
            dimension_semantics=("parallel",)),
    )(x_base, x_pa, mobility, ply2,
      w1x, w1m, b1, w2big, b2row, wol2, woxb, woxpa, borow)
    return out
